# baseline (speedup 1.0000x reference)
import functools

import jax
import jax.numpy as jnp
from jax import lax
from jax.experimental import pallas as pl
from jax.experimental.pallas import tpu as pltpu
from jax.experimental.pallas import tpu_sc as plsc

N = 10000
E = 320000
D = 128
VOCAB = 120
G = 64

NC = 2
NS = 16
NW = NC * NS
EPW = E // NW
CH = 40
STEPS = EPW // CH
WC = 50
NWIN = STEPS // WC
ACC_N = N
RPS = 624
RPS_LAST = N - RPS * (NS - 1)
RPS_LAST_Z = ACC_N - RPS * (NS - 1)

R = 1000
BN = N // R

def _mesh():
    return plsc.VectorSubcoreMesh(core_axis_name="c", subcore_axis_name="s")


def _zero_acc(zeros_h, acc, s):
    @pl.when(s < NS - 1)
    def _():
        pltpu.sync_copy(zeros_h.at[pl.ds(0, RPS)],
                        acc.at[pl.ds(s * RPS, RPS)])

    @pl.when(s == NS - 1)
    def _():
        pltpu.sync_copy(zeros_h, acc.at[pl.ds((NS - 1) * RPS, RPS_LAST_Z)])


def _dump_acc(acc, out0, out1, c, s):
    @pl.when(jnp.logical_and(c == 0, s < NS - 1))
    def _():
        pltpu.sync_copy(acc.at[pl.ds(s * RPS, RPS)],
                        out0.at[pl.ds(s * RPS, RPS)])

    @pl.when(jnp.logical_and(c == 0, s == NS - 1))
    def _():
        pltpu.sync_copy(acc.at[pl.ds((NS - 1) * RPS, RPS_LAST)],
                        out0.at[pl.ds((NS - 1) * RPS, RPS_LAST)])

    @pl.when(jnp.logical_and(c == 1, s < NS - 1))
    def _():
        pltpu.sync_copy(acc.at[pl.ds(s * RPS, RPS)],
                        out1.at[pl.ds(s * RPS, RPS)])

    @pl.when(jnp.logical_and(c == 1, s == NS - 1))
    def _():
        pltpu.sync_copy(acc.at[pl.ds((NS - 1) * RPS, RPS_LAST)],
                        out1.at[pl.ds((NS - 1) * RPS, RPS_LAST)])


def _sc_degree(dst3f, ones16, zeros16):
    @functools.partial(
        pl.kernel,
        mesh=_mesh(),
        out_type=[jax.ShapeDtypeStruct((N, 16), jnp.float32),
                  jax.ShapeDtypeStruct((N, 16), jnp.float32)],
        scratch_types=[
            pltpu.VMEM((STEPS, CH), jnp.int32),
            pltpu.VMEM((CH, 16), jnp.float32),
            pltpu.VMEM_SHARED((ACC_N, 16), jnp.float32),
        ],
    )
    def k(dst_h, ones_h, zeros_h, out0, out1, didx, ones_v, acc):
        c = lax.axis_index("c")
        s = lax.axis_index("s")
        wid = c * NS + s
        _zero_acc(zeros_h, acc, s)
        pltpu.sync_copy(ones_h, ones_v)
        pltpu.sync_copy(dst_h.at[wid], didx)
        plsc.subcore_barrier()

        def body(j, carry):
            pltpu.sync_copy(ones_v, acc.at[didx.at[j]], add=True)
            return carry

        lax.fori_loop(0, STEPS, body, 0)
        plsc.subcore_barrier()
        _dump_acc(acc, out0, out1, c, s)

    return k(dst3f, ones16, zeros16)


def _sc_edge_sum(table, src3, dst3, zeros_d):
    @functools.partial(
        pl.kernel,
        mesh=_mesh(),
        out_type=[jax.ShapeDtypeStruct((N, D), jnp.float32),
                  jax.ShapeDtypeStruct((N, D), jnp.float32)],
        scratch_types=[
            pltpu.VMEM((WC, CH), jnp.int32),
            pltpu.VMEM((WC, CH), jnp.int32),
            pltpu.VMEM((CH, D), jnp.float32),
            pltpu.VMEM((CH, D), jnp.float32),
            pltpu.VMEM_SHARED((ACC_N, D), jnp.float32),
            pltpu.SemaphoreType.DMA,
            pltpu.SemaphoreType.DMA,
        ],
    )
    def k(table_h, src_h, dst_h, zeros_h, out0, out1,
          sidx, didx, rows0, rows1, acc, sem0, sem1):
        c = lax.axis_index("c")
        s = lax.axis_index("s")
        wid = c * NS + s
        _zero_acc(zeros_h, acc, s)
        plsc.subcore_barrier()

        def body(j, carry):
            w = j // WC
            jw = j - w * WC

            @pl.when(jw == 0)
            def _():
                pltpu.sync_copy(src_h.at[wid * NWIN + w], sidx)
                pltpu.sync_copy(dst_h.at[wid * NWIN + w], didx)

            pltpu.async_copy(table_h.at[sidx.at[jw]], rows0, sem0).wait()
            pltpu.sync_copy(rows0, acc.at[didx.at[jw]], add=True)
            return carry

        lax.fori_loop(0, STEPS, body, 0)
        plsc.subcore_barrier()
        _dump_acc(acc, out0, out1, c, s)

    return k(table, src3, dst3, zeros_d)


def _dense_stage1(atoms3, deg0, deg1, embed_p, W1):
    def body(at_ref, d0_ref, d1_ref, emb_ref, w_ref, x0_ref, h_ref, dv_ref):
        at = at_ref[0, 0, :].reshape(R, 1)
        oh = (at == lax.broadcasted_iota(jnp.int32, (R, 128), 1)
              ).astype(jnp.float32)
        x0 = jnp.dot(oh, emb_ref[...], preferred_element_type=jnp.float32)
        deg = d0_ref[:, 0:1] + d1_ref[:, 0:1] + 1.0
        dinv = lax.rsqrt(deg)
        h = jnp.dot(x0, w_ref[...], preferred_element_type=jnp.float32) * dinv
        x0_ref[...] = x0
        h_ref[...] = h
        dv_ref[...] = dinv

    return pl.pallas_call(
        body,
        grid=(BN,),
        in_specs=[
            pl.BlockSpec((1, 1, R), lambda b: (b, 0, 0)),
            pl.BlockSpec((R, 16), lambda b: (b, 0)),
            pl.BlockSpec((R, 16), lambda b: (b, 0)),
            pl.BlockSpec((128, 128), lambda b: (0, 0)),
            pl.BlockSpec((128, 128), lambda b: (0, 0)),
        ],
        out_specs=[
            pl.BlockSpec((R, D), lambda b: (b, 0)),
            pl.BlockSpec((R, D), lambda b: (b, 0)),
            pl.BlockSpec((R, 1), lambda b: (b, 0)),
        ],
        out_shape=[
            jax.ShapeDtypeStruct((N, D), jnp.float32),
            jax.ShapeDtypeStruct((N, D), jnp.float32),
            jax.ShapeDtypeStruct((N, 1), jnp.float32),
        ],
    )(atoms3, deg0, deg1, embed_p, W1)


def _dense_conv(x, hp, s0, s1, dinv, bias, Wn):
    def body(x_ref, hp_ref, s0_ref, s1_ref, dv_ref, b_ref, w_ref,
             xn_ref, hn_ref):
        dv = dv_ref[...]
        conv = dv * (s0_ref[...] + s1_ref[...] + hp_ref[...]) + b_ref[...]
        xn = jnp.maximum(x_ref[...] + conv, 0.0)
        hn = jnp.dot(xn, w_ref[...], preferred_element_type=jnp.float32) * dv
        xn_ref[...] = xn
        hn_ref[...] = hn

    return pl.pallas_call(
        body,
        grid=(BN,),
        in_specs=[
            pl.BlockSpec((R, D), lambda b: (b, 0)),
            pl.BlockSpec((R, D), lambda b: (b, 0)),
            pl.BlockSpec((R, D), lambda b: (b, 0)),
            pl.BlockSpec((R, D), lambda b: (b, 0)),
            pl.BlockSpec((R, 1), lambda b: (b, 0)),
            pl.BlockSpec((1, D), lambda b: (0, 0)),
            pl.BlockSpec((128, 128), lambda b: (0, 0)),
        ],
        out_specs=[
            pl.BlockSpec((R, D), lambda b: (b, 0)),
            pl.BlockSpec((R, D), lambda b: (b, 0)),
        ],
        out_shape=[
            jax.ShapeDtypeStruct((N, D), jnp.float32),
            jax.ShapeDtypeStruct((N, D), jnp.float32),
        ],
    )(x, hp, s0, s1, dinv, bias, Wn)


def _dense_final(x, hp, s0, s1, dinv, bias, batch3, fcW, fcb2):
    def body(x_ref, hp_ref, s0_ref, s1_ref, dv_ref, b_ref, bt_ref,
             fw_ref, fb_ref, out_ref, S_ref, C_ref):
        i = pl.program_id(0)
        dv = dv_ref[...]
        conv = dv * (s0_ref[...] + s1_ref[...] + hp_ref[...]) + b_ref[...]
        xn = jnp.maximum(x_ref[...] + conv, 0.0)
        bt = bt_ref[0, 0, :].reshape(R, 1)
        oh = (bt == lax.broadcasted_iota(jnp.int32, (R, G), 1)
              ).astype(jnp.float32)
        contrib = lax.dot_general(oh, xn, (((0,), (0,)), ((), ())),
                                  preferred_element_type=jnp.float32)
        cnt = lax.dot_general(oh, jnp.ones((R, D), jnp.float32),
                              (((0,), (0,)), ((), ())),
                              preferred_element_type=jnp.float32)

        @pl.when(i == 0)
        def _():
            S_ref[...] = jnp.zeros((G, D), jnp.float32)
            C_ref[...] = jnp.zeros((G, D), jnp.float32)

        S_ref[...] += contrib
        C_ref[...] += cnt

        @pl.when(i == BN - 1)
        def _():
            pooled = S_ref[...] / jnp.maximum(C_ref[...], 1.0)
            out_ref[...] = (jnp.dot(pooled, fw_ref[...],
                                    preferred_element_type=jnp.float32)
                            + fb_ref[...])

    return pl.pallas_call(
        body,
        grid=(BN,),
        in_specs=[
            pl.BlockSpec((R, D), lambda b: (b, 0)),
            pl.BlockSpec((R, D), lambda b: (b, 0)),
            pl.BlockSpec((R, D), lambda b: (b, 0)),
            pl.BlockSpec((R, D), lambda b: (b, 0)),
            pl.BlockSpec((R, 1), lambda b: (b, 0)),
            pl.BlockSpec((1, D), lambda b: (0, 0)),
            pl.BlockSpec((1, 1, R), lambda b: (b, 0, 0)),
            pl.BlockSpec((D, 1), lambda b: (0, 0)),
            pl.BlockSpec((1, 1), lambda b: (0, 0)),
        ],
        out_specs=pl.BlockSpec((G, 1), lambda b: (0, 0)),
        out_shape=jax.ShapeDtypeStruct((G, 1), jnp.float32),
        scratch_shapes=[
            pltpu.VMEM((G, D), jnp.float32),
            pltpu.VMEM((G, D), jnp.float32),
        ],
    )(x, hp, s0, s1, dinv, bias, batch3, fcW, fcb2)


def kernel(atoms, edge_index, batch, embed, W1, b1, W2, b2, W3, b3, fcW, fcb):
    src3 = edge_index[0].astype(jnp.int32).reshape(NW * NWIN, WC, CH)
    dst3 = edge_index[1].astype(jnp.int32).reshape(NW * NWIN, WC, CH)
    atoms3 = atoms.astype(jnp.int32).reshape(BN, 1, R)
    batch3 = batch.astype(jnp.int32).reshape(BN, 1, R)
    embed_p = jnp.pad(embed, ((0, 128 - VOCAB), (0, 0)))
    ones16 = jnp.ones((CH, 16), jnp.float32)
    zeros16 = jnp.zeros((RPS_LAST_Z, 16), jnp.float32)
    zeros_d = jnp.zeros((RPS_LAST_Z, D), jnp.float32)
    b1r = b1.reshape(1, D)
    b2r = b2.reshape(1, D)
    b3r = b3.reshape(1, D)
    fcb2 = fcb.reshape(1, 1)

    deg0, deg1 = _sc_degree(dst3.reshape(NW, STEPS, CH), ones16, zeros16)
    x0, h1p, dinv = _dense_stage1(atoms3, deg0, deg1, embed_p, W1)
    s10, s11 = _sc_edge_sum(h1p, src3, dst3, zeros_d)
    x1, h2p = _dense_conv(x0, h1p, s10, s11, dinv, b1r, W2)
    s20, s21 = _sc_edge_sum(h2p, src3, dst3, zeros_d)
    x2, h3p = _dense_conv(x1, h2p, s20, s21, dinv, b2r, W3)
    s30, s31 = _sc_edge_sum(h3p, src3, dst3, zeros_d)
    return _dense_final(x2, h3p, s30, s31, dinv, b3r, batch3, fcW, fcb2)


# speedup vs baseline: 1.3794x; 1.3794x over previous
import functools

import jax
import jax.numpy as jnp
from jax import lax
from jax.experimental import pallas as pl
from jax.experimental.pallas import tpu as pltpu
from jax.experimental.pallas import tpu_sc as plsc

N = 10000
E = 320000
D = 128
VOCAB = 120
G = 64

NC = 2
NS = 16
NW = NC * NS
EPW = E // NW
CH = 40
STEPS = EPW // CH
WC = 50
NWIN = STEPS // WC
ACC_N = N
RPS = 624
RPS_LAST = N - RPS * (NS - 1)
RPS_LAST_Z = ACC_N - RPS * (NS - 1)

R = 1000
BN = N // R

def _mesh():
    return plsc.VectorSubcoreMesh(core_axis_name="c", subcore_axis_name="s")


def _zero_acc(zeros_h, acc, s):
    @pl.when(s < NS - 1)
    def _():
        pltpu.sync_copy(zeros_h.at[pl.ds(0, RPS)],
                        acc.at[pl.ds(s * RPS, RPS)])

    @pl.when(s == NS - 1)
    def _():
        pltpu.sync_copy(zeros_h, acc.at[pl.ds((NS - 1) * RPS, RPS_LAST_Z)])


def _dump_acc(acc, out0, out1, c, s):
    @pl.when(jnp.logical_and(c == 0, s < NS - 1))
    def _():
        pltpu.sync_copy(acc.at[pl.ds(s * RPS, RPS)],
                        out0.at[pl.ds(s * RPS, RPS)])

    @pl.when(jnp.logical_and(c == 0, s == NS - 1))
    def _():
        pltpu.sync_copy(acc.at[pl.ds((NS - 1) * RPS, RPS_LAST)],
                        out0.at[pl.ds((NS - 1) * RPS, RPS_LAST)])

    @pl.when(jnp.logical_and(c == 1, s < NS - 1))
    def _():
        pltpu.sync_copy(acc.at[pl.ds(s * RPS, RPS)],
                        out1.at[pl.ds(s * RPS, RPS)])

    @pl.when(jnp.logical_and(c == 1, s == NS - 1))
    def _():
        pltpu.sync_copy(acc.at[pl.ds((NS - 1) * RPS, RPS_LAST)],
                        out1.at[pl.ds((NS - 1) * RPS, RPS_LAST)])


def _sc_degree(dst3f, ones16, zeros16):
    @functools.partial(
        pl.kernel,
        mesh=_mesh(),
        out_type=[jax.ShapeDtypeStruct((N, 16), jnp.float32),
                  jax.ShapeDtypeStruct((N, 16), jnp.float32)],
        scratch_types=[
            pltpu.VMEM((STEPS, CH), jnp.int32),
            pltpu.VMEM((CH, 16), jnp.float32),
            pltpu.VMEM_SHARED((ACC_N, 16), jnp.float32),
        ],
    )
    def k(dst_h, ones_h, zeros_h, out0, out1, didx, ones_v, acc):
        c = lax.axis_index("c")
        s = lax.axis_index("s")
        wid = c * NS + s
        _zero_acc(zeros_h, acc, s)
        pltpu.sync_copy(ones_h, ones_v)
        pltpu.sync_copy(dst_h.at[wid], didx)
        plsc.subcore_barrier()

        def body(j, carry):
            pltpu.sync_copy(ones_v, acc.at[didx.at[j]], add=True)
            return carry

        lax.fori_loop(0, STEPS, body, 0)
        plsc.subcore_barrier()
        _dump_acc(acc, out0, out1, c, s)

    return k(dst3f, ones16, zeros16)


def _sc_edge_sum(table, src3, dst3, zeros_d):
    @functools.partial(
        pl.kernel,
        mesh=_mesh(),
        out_type=[jax.ShapeDtypeStruct((N, D), jnp.float32),
                  jax.ShapeDtypeStruct((N, D), jnp.float32)],
        scratch_types=[
            pltpu.VMEM((WC, CH), jnp.int32),
            pltpu.VMEM((WC, CH), jnp.int32),
            pltpu.VMEM((CH, D), jnp.float32),
            pltpu.VMEM((CH, D), jnp.float32),
            pltpu.VMEM_SHARED((ACC_N, D), jnp.float32),
            pltpu.SemaphoreType.DMA,
            pltpu.SemaphoreType.DMA,
        ],
    )
    def k(table_h, src_h, dst_h, zeros_h, out0, out1,
          sidx, didx, rows0, rows1, acc, sem0, sem1):
        c = lax.axis_index("c")
        s = lax.axis_index("s")
        wid = c * NS + s
        _zero_acc(zeros_h, acc, s)
        plsc.subcore_barrier()

        def win(w, carry):
            pltpu.sync_copy(src_h.at[wid * NWIN + w], sidx)
            pltpu.sync_copy(dst_h.at[wid * NWIN + w], didx)

            def body(j2, carry2):
                j = 2 * j2
                cp0 = pltpu.async_copy(table_h.at[sidx.at[j]], rows0, sem0)
                cp1 = pltpu.async_copy(table_h.at[sidx.at[j + 1]], rows1,
                                       sem1)
                cp0.wait()
                pltpu.sync_copy(rows0, acc.at[didx.at[j]], add=True)
                cp1.wait()
                pltpu.sync_copy(rows1, acc.at[didx.at[j + 1]], add=True)
                return carry2

            lax.fori_loop(0, WC // 2, body, 0)
            return carry

        lax.fori_loop(0, NWIN, win, 0)
        plsc.subcore_barrier()
        _dump_acc(acc, out0, out1, c, s)

    return k(table, src3, dst3, zeros_d)


def _dense_stage1(atoms3, deg0, deg1, embed_p, W1):
    def body(at_ref, d0_ref, d1_ref, emb_ref, w_ref, x0_ref, h_ref, dv_ref):
        at = at_ref[0, 0, :].reshape(R, 1)
        oh = (at == lax.broadcasted_iota(jnp.int32, (R, 128), 1)
              ).astype(jnp.float32)
        x0 = jnp.dot(oh, emb_ref[...], preferred_element_type=jnp.float32)
        deg = d0_ref[:, 0:1] + d1_ref[:, 0:1] + 1.0
        dinv = lax.rsqrt(deg)
        h = jnp.dot(x0, w_ref[...], preferred_element_type=jnp.float32) * dinv
        x0_ref[...] = x0
        h_ref[...] = h
        dv_ref[...] = dinv

    return pl.pallas_call(
        body,
        grid=(BN,),
        in_specs=[
            pl.BlockSpec((1, 1, R), lambda b: (b, 0, 0)),
            pl.BlockSpec((R, 16), lambda b: (b, 0)),
            pl.BlockSpec((R, 16), lambda b: (b, 0)),
            pl.BlockSpec((128, 128), lambda b: (0, 0)),
            pl.BlockSpec((128, 128), lambda b: (0, 0)),
        ],
        out_specs=[
            pl.BlockSpec((R, D), lambda b: (b, 0)),
            pl.BlockSpec((R, D), lambda b: (b, 0)),
            pl.BlockSpec((R, 1), lambda b: (b, 0)),
        ],
        out_shape=[
            jax.ShapeDtypeStruct((N, D), jnp.float32),
            jax.ShapeDtypeStruct((N, D), jnp.float32),
            jax.ShapeDtypeStruct((N, 1), jnp.float32),
        ],
    )(atoms3, deg0, deg1, embed_p, W1)


def _dense_conv(x, hp, s0, s1, dinv, bias, Wn):
    def body(x_ref, hp_ref, s0_ref, s1_ref, dv_ref, b_ref, w_ref,
             xn_ref, hn_ref):
        dv = dv_ref[...]
        conv = dv * (s0_ref[...] + s1_ref[...] + hp_ref[...]) + b_ref[...]
        xn = jnp.maximum(x_ref[...] + conv, 0.0)
        hn = jnp.dot(xn, w_ref[...], preferred_element_type=jnp.float32) * dv
        xn_ref[...] = xn
        hn_ref[...] = hn

    return pl.pallas_call(
        body,
        grid=(BN,),
        in_specs=[
            pl.BlockSpec((R, D), lambda b: (b, 0)),
            pl.BlockSpec((R, D), lambda b: (b, 0)),
            pl.BlockSpec((R, D), lambda b: (b, 0)),
            pl.BlockSpec((R, D), lambda b: (b, 0)),
            pl.BlockSpec((R, 1), lambda b: (b, 0)),
            pl.BlockSpec((1, D), lambda b: (0, 0)),
            pl.BlockSpec((128, 128), lambda b: (0, 0)),
        ],
        out_specs=[
            pl.BlockSpec((R, D), lambda b: (b, 0)),
            pl.BlockSpec((R, D), lambda b: (b, 0)),
        ],
        out_shape=[
            jax.ShapeDtypeStruct((N, D), jnp.float32),
            jax.ShapeDtypeStruct((N, D), jnp.float32),
        ],
    )(x, hp, s0, s1, dinv, bias, Wn)


def _dense_final(x, hp, s0, s1, dinv, bias, batch3, fcW, fcb2):
    def body(x_ref, hp_ref, s0_ref, s1_ref, dv_ref, b_ref, bt_ref,
             fw_ref, fb_ref, out_ref, S_ref, C_ref):
        i = pl.program_id(0)
        dv = dv_ref[...]
        conv = dv * (s0_ref[...] + s1_ref[...] + hp_ref[...]) + b_ref[...]
        xn = jnp.maximum(x_ref[...] + conv, 0.0)
        bt = bt_ref[0, 0, :].reshape(R, 1)
        oh = (bt == lax.broadcasted_iota(jnp.int32, (R, G), 1)
              ).astype(jnp.float32)
        contrib = lax.dot_general(oh, xn, (((0,), (0,)), ((), ())),
                                  preferred_element_type=jnp.float32)
        cnt = lax.dot_general(oh, jnp.ones((R, D), jnp.float32),
                              (((0,), (0,)), ((), ())),
                              preferred_element_type=jnp.float32)

        @pl.when(i == 0)
        def _():
            S_ref[...] = jnp.zeros((G, D), jnp.float32)
            C_ref[...] = jnp.zeros((G, D), jnp.float32)

        S_ref[...] += contrib
        C_ref[...] += cnt

        @pl.when(i == BN - 1)
        def _():
            pooled = S_ref[...] / jnp.maximum(C_ref[...], 1.0)
            out_ref[...] = (jnp.dot(pooled, fw_ref[...],
                                    preferred_element_type=jnp.float32)
                            + fb_ref[...])

    return pl.pallas_call(
        body,
        grid=(BN,),
        in_specs=[
            pl.BlockSpec((R, D), lambda b: (b, 0)),
            pl.BlockSpec((R, D), lambda b: (b, 0)),
            pl.BlockSpec((R, D), lambda b: (b, 0)),
            pl.BlockSpec((R, D), lambda b: (b, 0)),
            pl.BlockSpec((R, 1), lambda b: (b, 0)),
            pl.BlockSpec((1, D), lambda b: (0, 0)),
            pl.BlockSpec((1, 1, R), lambda b: (b, 0, 0)),
            pl.BlockSpec((D, 1), lambda b: (0, 0)),
            pl.BlockSpec((1, 1), lambda b: (0, 0)),
        ],
        out_specs=pl.BlockSpec((G, 1), lambda b: (0, 0)),
        out_shape=jax.ShapeDtypeStruct((G, 1), jnp.float32),
        scratch_shapes=[
            pltpu.VMEM((G, D), jnp.float32),
            pltpu.VMEM((G, D), jnp.float32),
        ],
    )(x, hp, s0, s1, dinv, bias, batch3, fcW, fcb2)


def kernel(atoms, edge_index, batch, embed, W1, b1, W2, b2, W3, b3, fcW, fcb):
    src3 = edge_index[0].astype(jnp.int32).reshape(NW * NWIN, WC, CH)
    dst3 = edge_index[1].astype(jnp.int32).reshape(NW * NWIN, WC, CH)
    atoms3 = atoms.astype(jnp.int32).reshape(BN, 1, R)
    batch3 = batch.astype(jnp.int32).reshape(BN, 1, R)
    embed_p = jnp.pad(embed, ((0, 128 - VOCAB), (0, 0)))
    ones16 = jnp.ones((CH, 16), jnp.float32)
    zeros16 = jnp.zeros((RPS_LAST_Z, 16), jnp.float32)
    zeros_d = jnp.zeros((RPS_LAST_Z, D), jnp.float32)
    b1r = b1.reshape(1, D)
    b2r = b2.reshape(1, D)
    b3r = b3.reshape(1, D)
    fcb2 = fcb.reshape(1, 1)

    deg0, deg1 = _sc_degree(dst3.reshape(NW, STEPS, CH), ones16, zeros16)
    x0, h1p, dinv = _dense_stage1(atoms3, deg0, deg1, embed_p, W1)
    s10, s11 = _sc_edge_sum(h1p, src3, dst3, zeros_d)
    x1, h2p = _dense_conv(x0, h1p, s10, s11, dinv, b1r, W2)
    s20, s21 = _sc_edge_sum(h2p, src3, dst3, zeros_d)
    x2, h3p = _dense_conv(x1, h2p, s20, s21, dinv, b2r, W3)
    s30, s31 = _sc_edge_sum(h3p, src3, dst3, zeros_d)
    return _dense_final(x2, h3p, s30, s31, dinv, b3r, batch3, fcW, fcb2)


# speedup vs baseline: 1.6683x; 1.2094x over previous
import functools

import jax
import jax.numpy as jnp
from jax import lax
from jax.experimental import pallas as pl
from jax.experimental.pallas import tpu as pltpu
from jax.experimental.pallas import tpu_sc as plsc

N = 10000
E = 320000
D = 128
VOCAB = 120
G = 64

NC = 2
NS = 16
NW = NC * NS
EPW = E // NW
CH = 40
STEPS = EPW // CH
WC = 50
NWIN = STEPS // WC
ACC_N = N
RPS = 624
RPS_LAST = N - RPS * (NS - 1)
RPS_LAST_Z = ACC_N - RPS * (NS - 1)

ECH = 100
ESTEPS = EPW // ECH
EWC = 20
ENWIN = ESTEPS // EWC

R = 1000
BN = N // R

def _mesh():
    return plsc.VectorSubcoreMesh(core_axis_name="c", subcore_axis_name="s")


def _zero_acc(zeros_h, acc, s):
    @pl.when(s < NS - 1)
    def _():
        pltpu.sync_copy(zeros_h.at[pl.ds(0, RPS)],
                        acc.at[pl.ds(s * RPS, RPS)])

    @pl.when(s == NS - 1)
    def _():
        pltpu.sync_copy(zeros_h, acc.at[pl.ds((NS - 1) * RPS, RPS_LAST_Z)])


def _dump_acc(acc, out0, out1, c, s):
    @pl.when(jnp.logical_and(c == 0, s < NS - 1))
    def _():
        pltpu.sync_copy(acc.at[pl.ds(s * RPS, RPS)],
                        out0.at[pl.ds(s * RPS, RPS)])

    @pl.when(jnp.logical_and(c == 0, s == NS - 1))
    def _():
        pltpu.sync_copy(acc.at[pl.ds((NS - 1) * RPS, RPS_LAST)],
                        out0.at[pl.ds((NS - 1) * RPS, RPS_LAST)])

    @pl.when(jnp.logical_and(c == 1, s < NS - 1))
    def _():
        pltpu.sync_copy(acc.at[pl.ds(s * RPS, RPS)],
                        out1.at[pl.ds(s * RPS, RPS)])

    @pl.when(jnp.logical_and(c == 1, s == NS - 1))
    def _():
        pltpu.sync_copy(acc.at[pl.ds((NS - 1) * RPS, RPS_LAST)],
                        out1.at[pl.ds((NS - 1) * RPS, RPS_LAST)])


def _sc_degree(dst3f, ones16, zeros16):
    @functools.partial(
        pl.kernel,
        mesh=_mesh(),
        out_type=[jax.ShapeDtypeStruct((N, 16), jnp.float32),
                  jax.ShapeDtypeStruct((N, 16), jnp.float32)],
        scratch_types=[
            pltpu.VMEM((STEPS, CH), jnp.int32),
            pltpu.VMEM((CH, 16), jnp.float32),
            pltpu.VMEM_SHARED((ACC_N, 16), jnp.float32),
        ],
    )
    def k(dst_h, ones_h, zeros_h, out0, out1, didx, ones_v, acc):
        c = lax.axis_index("c")
        s = lax.axis_index("s")
        wid = c * NS + s
        _zero_acc(zeros_h, acc, s)
        pltpu.sync_copy(ones_h, ones_v)
        pltpu.sync_copy(dst_h.at[wid], didx)
        plsc.subcore_barrier()

        def body(j, carry):
            pltpu.sync_copy(ones_v, acc.at[didx.at[j]], add=True)
            return carry

        lax.fori_loop(0, STEPS, body, 0)
        plsc.subcore_barrier()
        _dump_acc(acc, out0, out1, c, s)

    return k(dst3f, ones16, zeros16)


def _sc_edge_sum(table, src3, dst3, zeros_d):
    @functools.partial(
        pl.kernel,
        mesh=_mesh(),
        out_type=[jax.ShapeDtypeStruct((N, D), jnp.float32),
                  jax.ShapeDtypeStruct((N, D), jnp.float32)],
        scratch_types=[
            pltpu.VMEM((EWC, ECH), jnp.int32),
            pltpu.VMEM((EWC, ECH), jnp.int32),
            pltpu.VMEM((ECH, D), jnp.float32),
            pltpu.VMEM((ECH, D), jnp.float32),
            pltpu.VMEM_SHARED((ACC_N, D), jnp.float32),
            pltpu.SemaphoreType.DMA,
            pltpu.SemaphoreType.DMA,
        ],
    )
    def k(table_h, src_h, dst_h, zeros_h, out0, out1,
          sidx, didx, rows0, rows1, acc, sem0, sem1):
        c = lax.axis_index("c")
        s = lax.axis_index("s")
        wid = c * NS + s
        _zero_acc(zeros_h, acc, s)
        plsc.subcore_barrier()

        def win(w, carry):
            pltpu.sync_copy(src_h.at[wid * ENWIN + w], sidx)
            pltpu.sync_copy(dst_h.at[wid * ENWIN + w], didx)

            def body(j2, carry2):
                j = 2 * j2
                cp0 = pltpu.async_copy(table_h.at[sidx.at[j]], rows0, sem0)
                cp1 = pltpu.async_copy(table_h.at[sidx.at[j + 1]], rows1,
                                       sem1)
                cp0.wait()
                pltpu.sync_copy(rows0, acc.at[didx.at[j]], add=True)
                cp1.wait()
                pltpu.sync_copy(rows1, acc.at[didx.at[j + 1]], add=True)
                return carry2

            lax.fori_loop(0, EWC // 2, body, 0)
            return carry

        lax.fori_loop(0, ENWIN, win, 0)
        plsc.subcore_barrier()
        _dump_acc(acc, out0, out1, c, s)

    return k(table, src3, dst3, zeros_d)


def _dense_stage1(atoms3, deg0, deg1, embed_p, W1):
    def body(at_ref, d0_ref, d1_ref, emb_ref, w_ref, x0_ref, h_ref, dv_ref):
        at = at_ref[0, 0, :].reshape(R, 1)
        oh = (at == lax.broadcasted_iota(jnp.int32, (R, 128), 1)
              ).astype(jnp.float32)
        x0 = jnp.dot(oh, emb_ref[...], preferred_element_type=jnp.float32)
        deg = d0_ref[:, 0:1] + d1_ref[:, 0:1] + 1.0
        dinv = lax.rsqrt(deg)
        h = jnp.dot(x0, w_ref[...], preferred_element_type=jnp.float32) * dinv
        x0_ref[...] = x0
        h_ref[...] = h
        dv_ref[...] = dinv

    return pl.pallas_call(
        body,
        grid=(BN,),
        in_specs=[
            pl.BlockSpec((1, 1, R), lambda b: (b, 0, 0)),
            pl.BlockSpec((R, 16), lambda b: (b, 0)),
            pl.BlockSpec((R, 16), lambda b: (b, 0)),
            pl.BlockSpec((128, 128), lambda b: (0, 0)),
            pl.BlockSpec((128, 128), lambda b: (0, 0)),
        ],
        out_specs=[
            pl.BlockSpec((R, D), lambda b: (b, 0)),
            pl.BlockSpec((R, D), lambda b: (b, 0)),
            pl.BlockSpec((R, 1), lambda b: (b, 0)),
        ],
        out_shape=[
            jax.ShapeDtypeStruct((N, D), jnp.float32),
            jax.ShapeDtypeStruct((N, D), jnp.float32),
            jax.ShapeDtypeStruct((N, 1), jnp.float32),
        ],
    )(atoms3, deg0, deg1, embed_p, W1)


def _dense_conv(x, hp, s0, s1, dinv, bias, Wn):
    def body(x_ref, hp_ref, s0_ref, s1_ref, dv_ref, b_ref, w_ref,
             xn_ref, hn_ref):
        dv = dv_ref[...]
        conv = dv * (s0_ref[...] + s1_ref[...] + hp_ref[...]) + b_ref[...]
        xn = jnp.maximum(x_ref[...] + conv, 0.0)
        hn = jnp.dot(xn, w_ref[...], preferred_element_type=jnp.float32) * dv
        xn_ref[...] = xn
        hn_ref[...] = hn

    return pl.pallas_call(
        body,
        grid=(BN,),
        in_specs=[
            pl.BlockSpec((R, D), lambda b: (b, 0)),
            pl.BlockSpec((R, D), lambda b: (b, 0)),
            pl.BlockSpec((R, D), lambda b: (b, 0)),
            pl.BlockSpec((R, D), lambda b: (b, 0)),
            pl.BlockSpec((R, 1), lambda b: (b, 0)),
            pl.BlockSpec((1, D), lambda b: (0, 0)),
            pl.BlockSpec((128, 128), lambda b: (0, 0)),
        ],
        out_specs=[
            pl.BlockSpec((R, D), lambda b: (b, 0)),
            pl.BlockSpec((R, D), lambda b: (b, 0)),
        ],
        out_shape=[
            jax.ShapeDtypeStruct((N, D), jnp.float32),
            jax.ShapeDtypeStruct((N, D), jnp.float32),
        ],
    )(x, hp, s0, s1, dinv, bias, Wn)


def _dense_final(x, hp, s0, s1, dinv, bias, batch3, fcW, fcb2):
    def body(x_ref, hp_ref, s0_ref, s1_ref, dv_ref, b_ref, bt_ref,
             fw_ref, fb_ref, out_ref, S_ref, C_ref):
        i = pl.program_id(0)
        dv = dv_ref[...]
        conv = dv * (s0_ref[...] + s1_ref[...] + hp_ref[...]) + b_ref[...]
        xn = jnp.maximum(x_ref[...] + conv, 0.0)
        bt = bt_ref[0, 0, :].reshape(R, 1)
        oh = (bt == lax.broadcasted_iota(jnp.int32, (R, G), 1)
              ).astype(jnp.float32)
        contrib = lax.dot_general(oh, xn, (((0,), (0,)), ((), ())),
                                  preferred_element_type=jnp.float32)
        cnt = lax.dot_general(oh, jnp.ones((R, D), jnp.float32),
                              (((0,), (0,)), ((), ())),
                              preferred_element_type=jnp.float32)

        @pl.when(i == 0)
        def _():
            S_ref[...] = jnp.zeros((G, D), jnp.float32)
            C_ref[...] = jnp.zeros((G, D), jnp.float32)

        S_ref[...] += contrib
        C_ref[...] += cnt

        @pl.when(i == BN - 1)
        def _():
            pooled = S_ref[...] / jnp.maximum(C_ref[...], 1.0)
            out_ref[...] = (jnp.dot(pooled, fw_ref[...],
                                    preferred_element_type=jnp.float32)
                            + fb_ref[...])

    return pl.pallas_call(
        body,
        grid=(BN,),
        in_specs=[
            pl.BlockSpec((R, D), lambda b: (b, 0)),
            pl.BlockSpec((R, D), lambda b: (b, 0)),
            pl.BlockSpec((R, D), lambda b: (b, 0)),
            pl.BlockSpec((R, D), lambda b: (b, 0)),
            pl.BlockSpec((R, 1), lambda b: (b, 0)),
            pl.BlockSpec((1, D), lambda b: (0, 0)),
            pl.BlockSpec((1, 1, R), lambda b: (b, 0, 0)),
            pl.BlockSpec((D, 1), lambda b: (0, 0)),
            pl.BlockSpec((1, 1), lambda b: (0, 0)),
        ],
        out_specs=pl.BlockSpec((G, 1), lambda b: (0, 0)),
        out_shape=jax.ShapeDtypeStruct((G, 1), jnp.float32),
        scratch_shapes=[
            pltpu.VMEM((G, D), jnp.float32),
            pltpu.VMEM((G, D), jnp.float32),
        ],
    )(x, hp, s0, s1, dinv, bias, batch3, fcW, fcb2)


def kernel(atoms, edge_index, batch, embed, W1, b1, W2, b2, W3, b3, fcW, fcb):
    src3 = edge_index[0].astype(jnp.int32).reshape(NW * ENWIN, EWC, ECH)
    dst3 = edge_index[1].astype(jnp.int32).reshape(NW * ENWIN, EWC, ECH)
    dstd = edge_index[1].astype(jnp.int32).reshape(NW, STEPS, CH)
    atoms3 = atoms.astype(jnp.int32).reshape(BN, 1, R)
    batch3 = batch.astype(jnp.int32).reshape(BN, 1, R)
    embed_p = jnp.pad(embed, ((0, 128 - VOCAB), (0, 0)))
    ones16 = jnp.ones((CH, 16), jnp.float32)
    zeros16 = jnp.zeros((RPS_LAST_Z, 16), jnp.float32)
    zeros_d = jnp.zeros((RPS_LAST_Z, D), jnp.float32)
    b1r = b1.reshape(1, D)
    b2r = b2.reshape(1, D)
    b3r = b3.reshape(1, D)
    fcb2 = fcb.reshape(1, 1)

    deg0, deg1 = _sc_degree(dstd, ones16, zeros16)
    x0, h1p, dinv = _dense_stage1(atoms3, deg0, deg1, embed_p, W1)
    s10, s11 = _sc_edge_sum(h1p, src3, dst3, zeros_d)
    x1, h2p = _dense_conv(x0, h1p, s10, s11, dinv, b1r, W2)
    s20, s21 = _sc_edge_sum(h2p, src3, dst3, zeros_d)
    x2, h3p = _dense_conv(x1, h2p, s20, s21, dinv, b2r, W3)
    s30, s31 = _sc_edge_sum(h3p, src3, dst3, zeros_d)
    return _dense_final(x2, h3p, s30, s31, dinv, b3r, batch3, fcW, fcb2)


# speedup vs baseline: 1.7608x; 1.0554x over previous
import functools

import jax
import jax.numpy as jnp
from jax import lax
from jax.experimental import pallas as pl
from jax.experimental.pallas import tpu as pltpu
from jax.experimental.pallas import tpu_sc as plsc

N = 10000
E = 320000
D = 128
VOCAB = 120
G = 64

NC = 2
NS = 16
NW = NC * NS
EPW = E // NW
CH = 100
STEPS = EPW // CH
ACC_N = N
RPS = 624
RPS_LAST = N - RPS * (NS - 1)
RPS_LAST_Z = ACC_N - RPS * (NS - 1)

ECH = 125
ESTEPS = EPW // ECH
EWC = 16
ENWIN = ESTEPS // EWC

R = 1000
BN = N // R

def _mesh():
    return plsc.VectorSubcoreMesh(core_axis_name="c", subcore_axis_name="s")


def _zero_acc(zeros_h, acc, s):
    @pl.when(s < NS - 1)
    def _():
        pltpu.sync_copy(zeros_h.at[pl.ds(0, RPS)],
                        acc.at[pl.ds(s * RPS, RPS)])

    @pl.when(s == NS - 1)
    def _():
        pltpu.sync_copy(zeros_h, acc.at[pl.ds((NS - 1) * RPS, RPS_LAST_Z)])


def _dump_acc(acc, out0, out1, c, s):
    @pl.when(jnp.logical_and(c == 0, s < NS - 1))
    def _():
        pltpu.sync_copy(acc.at[pl.ds(s * RPS, RPS)],
                        out0.at[pl.ds(s * RPS, RPS)])

    @pl.when(jnp.logical_and(c == 0, s == NS - 1))
    def _():
        pltpu.sync_copy(acc.at[pl.ds((NS - 1) * RPS, RPS_LAST)],
                        out0.at[pl.ds((NS - 1) * RPS, RPS_LAST)])

    @pl.when(jnp.logical_and(c == 1, s < NS - 1))
    def _():
        pltpu.sync_copy(acc.at[pl.ds(s * RPS, RPS)],
                        out1.at[pl.ds(s * RPS, RPS)])

    @pl.when(jnp.logical_and(c == 1, s == NS - 1))
    def _():
        pltpu.sync_copy(acc.at[pl.ds((NS - 1) * RPS, RPS_LAST)],
                        out1.at[pl.ds((NS - 1) * RPS, RPS_LAST)])


def _sc_degree(dst3f, ones16, zeros16):
    @functools.partial(
        pl.kernel,
        mesh=_mesh(),
        out_type=[jax.ShapeDtypeStruct((N, 16), jnp.float32),
                  jax.ShapeDtypeStruct((N, 16), jnp.float32)],
        scratch_types=[
            pltpu.VMEM((STEPS, CH), jnp.int32),
            pltpu.VMEM((CH, 16), jnp.float32),
            pltpu.VMEM_SHARED((ACC_N, 16), jnp.float32),
        ],
    )
    def k(dst_h, ones_h, zeros_h, out0, out1, didx, ones_v, acc):
        c = lax.axis_index("c")
        s = lax.axis_index("s")
        wid = c * NS + s
        _zero_acc(zeros_h, acc, s)
        pltpu.sync_copy(ones_h, ones_v)
        pltpu.sync_copy(dst_h.at[wid], didx)
        plsc.subcore_barrier()

        def body(j, carry):
            pltpu.sync_copy(ones_v, acc.at[didx.at[j]], add=True)
            return carry

        lax.fori_loop(0, STEPS, body, 0)
        plsc.subcore_barrier()
        _dump_acc(acc, out0, out1, c, s)

    return k(dst3f, ones16, zeros16)


def _sc_edge_sum(table, src3, dst3, zeros_d):
    @functools.partial(
        pl.kernel,
        mesh=_mesh(),
        out_type=[jax.ShapeDtypeStruct((N, D), jnp.float32),
                  jax.ShapeDtypeStruct((N, D), jnp.float32)],
        scratch_types=[
            pltpu.VMEM((EWC, ECH), jnp.int32),
            pltpu.VMEM((EWC, ECH), jnp.int32),
            pltpu.VMEM((ECH, D), jnp.float32),
            pltpu.VMEM((ECH, D), jnp.float32),
            pltpu.VMEM_SHARED((ACC_N, D), jnp.float32),
            pltpu.SemaphoreType.DMA,
            pltpu.SemaphoreType.DMA,
        ],
    )
    def k(table_h, src_h, dst_h, zeros_h, out0, out1,
          sidx, didx, rows0, rows1, acc, sem0, sem1):
        c = lax.axis_index("c")
        s = lax.axis_index("s")
        wid = c * NS + s
        _zero_acc(zeros_h, acc, s)
        plsc.subcore_barrier()

        def win(w, carry):
            pltpu.sync_copy(src_h.at[wid * ENWIN + w], sidx)
            pltpu.sync_copy(dst_h.at[wid * ENWIN + w], didx)

            def body(j2, carry2):
                j = 2 * j2
                cp0 = pltpu.async_copy(table_h.at[sidx.at[j]], rows0, sem0)
                cp1 = pltpu.async_copy(table_h.at[sidx.at[j + 1]], rows1,
                                       sem1)
                cp0.wait()
                pltpu.sync_copy(rows0, acc.at[didx.at[j]], add=True)
                cp1.wait()
                pltpu.sync_copy(rows1, acc.at[didx.at[j + 1]], add=True)
                return carry2

            lax.fori_loop(0, EWC // 2, body, 0)
            return carry

        lax.fori_loop(0, ENWIN, win, 0)
        plsc.subcore_barrier()
        _dump_acc(acc, out0, out1, c, s)

    return k(table, src3, dst3, zeros_d)


def _dense_stage1(atoms3, deg0, deg1, embed_p, W1):
    def body(at_ref, d0_ref, d1_ref, emb_ref, w_ref, x0_ref, h_ref, dv_ref):
        at = at_ref[0, 0, :].reshape(R, 1)
        oh = (at == lax.broadcasted_iota(jnp.int32, (R, 128), 1)
              ).astype(jnp.float32)
        x0 = jnp.dot(oh, emb_ref[...], preferred_element_type=jnp.float32)
        deg = d0_ref[:, 0:1] + d1_ref[:, 0:1] + 1.0
        dinv = lax.rsqrt(deg)
        h = jnp.dot(x0, w_ref[...], preferred_element_type=jnp.float32) * dinv
        x0_ref[...] = x0
        h_ref[...] = h
        dv_ref[...] = dinv

    return pl.pallas_call(
        body,
        grid=(BN,),
        in_specs=[
            pl.BlockSpec((1, 1, R), lambda b: (b, 0, 0)),
            pl.BlockSpec((R, 16), lambda b: (b, 0)),
            pl.BlockSpec((R, 16), lambda b: (b, 0)),
            pl.BlockSpec((128, 128), lambda b: (0, 0)),
            pl.BlockSpec((128, 128), lambda b: (0, 0)),
        ],
        out_specs=[
            pl.BlockSpec((R, D), lambda b: (b, 0)),
            pl.BlockSpec((R, D), lambda b: (b, 0)),
            pl.BlockSpec((R, 1), lambda b: (b, 0)),
        ],
        out_shape=[
            jax.ShapeDtypeStruct((N, D), jnp.float32),
            jax.ShapeDtypeStruct((N, D), jnp.float32),
            jax.ShapeDtypeStruct((N, 1), jnp.float32),
        ],
    )(atoms3, deg0, deg1, embed_p, W1)


def _dense_conv(x, hp, s0, s1, dinv, bias, Wn):
    def body(x_ref, hp_ref, s0_ref, s1_ref, dv_ref, b_ref, w_ref,
             xn_ref, hn_ref):
        dv = dv_ref[...]
        conv = dv * (s0_ref[...] + s1_ref[...] + hp_ref[...]) + b_ref[...]
        xn = jnp.maximum(x_ref[...] + conv, 0.0)
        hn = jnp.dot(xn, w_ref[...], preferred_element_type=jnp.float32) * dv
        xn_ref[...] = xn
        hn_ref[...] = hn

    return pl.pallas_call(
        body,
        grid=(BN,),
        in_specs=[
            pl.BlockSpec((R, D), lambda b: (b, 0)),
            pl.BlockSpec((R, D), lambda b: (b, 0)),
            pl.BlockSpec((R, D), lambda b: (b, 0)),
            pl.BlockSpec((R, D), lambda b: (b, 0)),
            pl.BlockSpec((R, 1), lambda b: (b, 0)),
            pl.BlockSpec((1, D), lambda b: (0, 0)),
            pl.BlockSpec((128, 128), lambda b: (0, 0)),
        ],
        out_specs=[
            pl.BlockSpec((R, D), lambda b: (b, 0)),
            pl.BlockSpec((R, D), lambda b: (b, 0)),
        ],
        out_shape=[
            jax.ShapeDtypeStruct((N, D), jnp.float32),
            jax.ShapeDtypeStruct((N, D), jnp.float32),
        ],
    )(x, hp, s0, s1, dinv, bias, Wn)


def _dense_final(x, hp, s0, s1, dinv, bias, batch3, fcW, fcb2):
    def body(x_ref, hp_ref, s0_ref, s1_ref, dv_ref, b_ref, bt_ref,
             fw_ref, fb_ref, out_ref, S_ref, C_ref):
        i = pl.program_id(0)
        dv = dv_ref[...]
        conv = dv * (s0_ref[...] + s1_ref[...] + hp_ref[...]) + b_ref[...]
        xn = jnp.maximum(x_ref[...] + conv, 0.0)
        bt = bt_ref[0, 0, :].reshape(R, 1)
        oh = (bt == lax.broadcasted_iota(jnp.int32, (R, G), 1)
              ).astype(jnp.float32)
        contrib = lax.dot_general(oh, xn, (((0,), (0,)), ((), ())),
                                  preferred_element_type=jnp.float32)
        cnt = lax.dot_general(oh, jnp.ones((R, D), jnp.float32),
                              (((0,), (0,)), ((), ())),
                              preferred_element_type=jnp.float32)

        @pl.when(i == 0)
        def _():
            S_ref[...] = jnp.zeros((G, D), jnp.float32)
            C_ref[...] = jnp.zeros((G, D), jnp.float32)

        S_ref[...] += contrib
        C_ref[...] += cnt

        @pl.when(i == BN - 1)
        def _():
            pooled = S_ref[...] / jnp.maximum(C_ref[...], 1.0)
            out_ref[...] = (jnp.dot(pooled, fw_ref[...],
                                    preferred_element_type=jnp.float32)
                            + fb_ref[...])

    return pl.pallas_call(
        body,
        grid=(BN,),
        in_specs=[
            pl.BlockSpec((R, D), lambda b: (b, 0)),
            pl.BlockSpec((R, D), lambda b: (b, 0)),
            pl.BlockSpec((R, D), lambda b: (b, 0)),
            pl.BlockSpec((R, D), lambda b: (b, 0)),
            pl.BlockSpec((R, 1), lambda b: (b, 0)),
            pl.BlockSpec((1, D), lambda b: (0, 0)),
            pl.BlockSpec((1, 1, R), lambda b: (b, 0, 0)),
            pl.BlockSpec((D, 1), lambda b: (0, 0)),
            pl.BlockSpec((1, 1), lambda b: (0, 0)),
        ],
        out_specs=pl.BlockSpec((G, 1), lambda b: (0, 0)),
        out_shape=jax.ShapeDtypeStruct((G, 1), jnp.float32),
        scratch_shapes=[
            pltpu.VMEM((G, D), jnp.float32),
            pltpu.VMEM((G, D), jnp.float32),
        ],
    )(x, hp, s0, s1, dinv, bias, batch3, fcW, fcb2)


def kernel(atoms, edge_index, batch, embed, W1, b1, W2, b2, W3, b3, fcW, fcb):
    src3 = edge_index[0].astype(jnp.int32).reshape(NW * ENWIN, EWC, ECH)
    dst3 = edge_index[1].astype(jnp.int32).reshape(NW * ENWIN, EWC, ECH)
    dstd = edge_index[1].astype(jnp.int32).reshape(NW, STEPS, CH)
    atoms3 = atoms.astype(jnp.int32).reshape(BN, 1, R)
    batch3 = batch.astype(jnp.int32).reshape(BN, 1, R)
    embed_p = jnp.pad(embed, ((0, 128 - VOCAB), (0, 0)))
    ones16 = jnp.ones((CH, 16), jnp.float32)
    zeros16 = jnp.zeros((RPS_LAST_Z, 16), jnp.float32)
    zeros_d = jnp.zeros((RPS_LAST_Z, D), jnp.float32)
    b1r = b1.reshape(1, D)
    b2r = b2.reshape(1, D)
    b3r = b3.reshape(1, D)
    fcb2 = fcb.reshape(1, 1)

    deg0, deg1 = _sc_degree(dstd, ones16, zeros16)
    x0, h1p, dinv = _dense_stage1(atoms3, deg0, deg1, embed_p, W1)
    s10, s11 = _sc_edge_sum(h1p, src3, dst3, zeros_d)
    x1, h2p = _dense_conv(x0, h1p, s10, s11, dinv, b1r, W2)
    s20, s21 = _sc_edge_sum(h2p, src3, dst3, zeros_d)
    x2, h3p = _dense_conv(x1, h2p, s20, s21, dinv, b2r, W3)
    s30, s31 = _sc_edge_sum(h3p, src3, dst3, zeros_d)
    return _dense_final(x2, h3p, s30, s31, dinv, b3r, batch3, fcW, fcb2)
